# baseline (speedup 1.0000x reference)
import jax, jax.numpy as jnp
from jax.experimental import pallas as pl

def kernel(word, posh, post, word_table, pos1_table, pos2_table):
    w = jnp.take(word_table, word, axis=0)
    p1 = jnp.take(pos1_table, posh, axis=0)
    p2 = jnp.take(pos2_table, post, axis=0)
    return jnp.concatenate([w, p1, p2], axis=2)


# speedup vs baseline: 1.2620x; 1.2620x over previous
import functools

import jax
import jax.numpy as jnp
from jax import lax
from jax.experimental import pallas as pl
from jax.experimental.pallas import tpu as pltpu
from jax.experimental.pallas import tpu_sc as plsc

B = 1024
L = 200
N = B * L
DW = 300
DP = 5
DOUT = DW + 2 * DP
PTAB = 2 * 200 * 5

NC = 2
NS = 16
NW = NC * NS
ROWS_PER_TILE = N // NW
BLK = 128
NBLK = ROWS_PER_TILE // BLK
LANES = 16
DWP = 304


def _body(word_hbm, posh_hbm, post_hbm, wtab_hbm, p1_hbm, p2_hbm, out_hbm,
          widx, phidx, ptidx, wordbuf, outbuf, p1v, p2v, sem):
    c = lax.axis_index("c")
    s = lax.axis_index("s")
    wid = s * NC + c
    tile_base = wid * ROWS_PER_TILE

    pltpu.sync_copy(p1_hbm, p1v)
    pltpu.sync_copy(p2_hbm, p2v)

    def block(g, carry):
        base = tile_base + g * BLK
        pltpu.sync_copy(word_hbm.at[pl.ds(base, BLK)], widx)
        gather = pltpu.async_copy(wtab_hbm.at[widx], wordbuf, sem)
        pltpu.sync_copy(posh_hbm.at[pl.ds(base, BLK)], phidx)
        pltpu.sync_copy(post_hbm.at[pl.ds(base, BLK)], ptidx)

        gather.wait()

        def row_copy(r, carry2):
            for col in range(0, DWP, LANES):
                outbuf[r, pl.ds(col, LANES)] = wordbuf[r, pl.ds(col, LANES)]
            return carry2

        lax.fori_loop(0, BLK, row_copy, 0)

        for i in range(BLK // LANES):
            rows = lax.iota(jnp.int32, LANES) + i * LANES
            ph = phidx[pl.ds(i * LANES, LANES)] * DP
            pt = ptidx[pl.ds(i * LANES, LANES)] * DP
            for j in range(DP):
                v1 = plsc.load_gather(p1v, [ph + j])
                plsc.store_scatter(
                    outbuf, [rows, jnp.full((LANES,), DW + j, jnp.int32)], v1)
                v2 = plsc.load_gather(p2v, [pt + j])
                plsc.store_scatter(
                    outbuf, [rows, jnp.full((LANES,), DW + DP + j, jnp.int32)],
                    v2)

        pltpu.sync_copy(outbuf, out_hbm.at[pl.ds(base, BLK), :])
        return carry

    lax.fori_loop(0, NBLK, block, 0)


@jax.jit
def _run(word_flat, posh_flat, post_flat, word_table, p1_flat, p2_flat):
    mesh = plsc.VectorSubcoreMesh(
        core_axis_name="c", subcore_axis_name="s",
        num_cores=NC, num_subcores=NS)
    return pl.kernel(
        _body,
        out_type=jax.ShapeDtypeStruct((N, DOUT), jnp.float32),
        mesh=mesh,
        compiler_params=pltpu.CompilerParams(
            use_tc_tiling_on_sc=False, needs_layout_passes=False),
        scratch_types=[
            pltpu.VMEM((BLK,), jnp.int32),
            pltpu.VMEM((BLK,), jnp.int32),
            pltpu.VMEM((BLK,), jnp.int32),
            pltpu.VMEM((BLK, DWP), jnp.float32),
            pltpu.VMEM((BLK, DOUT), jnp.float32),
            pltpu.VMEM((PTAB,), jnp.float32),
            pltpu.VMEM((PTAB,), jnp.float32),
            pltpu.SemaphoreType.DMA,
        ],
    )(word_flat, posh_flat, post_flat, word_table, p1_flat, p2_flat)


def kernel(word, posh, post, word_table, pos1_table, pos2_table):
    wf = word.reshape(N).astype(jnp.int32)
    ph = posh.reshape(N).astype(jnp.int32)
    pt = post.reshape(N).astype(jnp.int32)
    wt = jnp.pad(word_table, ((0, 0), (0, DWP - DW)))
    p1 = pos1_table.reshape(PTAB)
    p2 = pos2_table.reshape(PTAB)
    out = _run(wf, ph, pt, wt, p1, p2)
    return out.reshape(B, L, DOUT)


# speedup vs baseline: 1.4149x; 1.1212x over previous
import jax
import jax.numpy as jnp
from jax import lax
from jax.experimental import pallas as pl
from jax.experimental.pallas import tpu as pltpu
from jax.experimental.pallas import tpu_sc as plsc

B = 1024
L = 200
N = B * L
DW = 300
DP = 5
DOUT = DW + 2 * DP
DWP = 304
PTAB = 2 * 200 * 5

NC = 2
NS = 16
NW = NC * NS
ROWS_PER_TILE = N // NW
BLK = 64
NBLK = ROWS_PER_TILE // BLK
LANES = 16


def _assemble(wordbuf, outbuf, ph_all, pt_all, p1v, p2v, off):
    def row_copy(r, carry2):
        for col in range(0, DWP, LANES):
            outbuf[r, pl.ds(col, LANES)] = wordbuf[r, pl.ds(col, LANES)]
        return carry2

    lax.fori_loop(0, BLK, row_copy, 0)

    for i in range(BLK // LANES):
        rows = lax.iota(jnp.int32, LANES) + i * LANES
        ph = ph_all[pl.ds(off + i * LANES, LANES)] * DP
        pt = pt_all[pl.ds(off + i * LANES, LANES)] * DP
        for j in range(DP):
            v1 = plsc.load_gather(p1v, [ph + j])
            plsc.store_scatter(
                outbuf, [rows, jnp.full((LANES,), DW + j, jnp.int32)], v1)
            v2 = plsc.load_gather(p2v, [pt + j])
            plsc.store_scatter(
                outbuf, [rows, jnp.full((LANES,), DW + DP + j, jnp.int32)], v2)


def _body(word_hbm, posh_hbm, post_hbm, wtab_hbm, p1_hbm, p2_hbm, out_hbm,
          widx, ph_all, pt_all, p1v, p2v, wb0, wb1, ob0, ob1,
          gsem0, gsem1, wsem0, wsem1):
    wid = lax.axis_index("s") * NC + lax.axis_index("c")
    tile_base = wid * ROWS_PER_TILE

    pltpu.sync_copy(p1_hbm, p1v)
    pltpu.sync_copy(p2_hbm, p2v)
    pltpu.sync_copy(word_hbm.at[pl.ds(tile_base, ROWS_PER_TILE)], widx)
    pltpu.sync_copy(posh_hbm.at[pl.ds(tile_base, ROWS_PER_TILE)], ph_all)
    pltpu.sync_copy(post_hbm.at[pl.ds(tile_base, ROWS_PER_TILE)], pt_all)

    wbs = (wb0, wb1)
    obs = (ob0, ob1)
    gsems = (gsem0, gsem1)
    wsems = (wsem0, wsem1)

    pltpu.async_copy(wtab_hbm.at[widx.at[pl.ds(0, BLK)]], wb0, gsem0)

    def pair(k, carry):
        for half in (0, 1):
            g = 2 * k + half

            @pl.when(g + 1 < NBLK)
            def _prefetch():
                pltpu.async_copy(
                    wtab_hbm.at[widx.at[pl.ds((g + 1) * BLK, BLK)]],
                    wbs[1 - half], gsems[1 - half])

            pltpu.make_async_copy(
                wtab_hbm.at[widx.at[pl.ds(g * BLK, BLK)]],
                wbs[half], gsems[half]).wait()

            @pl.when(g >= 2)
            def _drain_prev_write():
                pltpu.make_async_copy(
                    obs[half],
                    out_hbm.at[pl.ds(tile_base + (g - 2) * BLK, BLK), :],
                    wsems[half]).wait()

            _assemble(wbs[half], obs[half], ph_all, pt_all, p1v, p2v, g * BLK)
            pltpu.async_copy(
                obs[half], out_hbm.at[pl.ds(tile_base + g * BLK, BLK), :],
                wsems[half])
        return carry

    lax.fori_loop(0, NBLK // 2, pair, 0)

    pltpu.make_async_copy(
        ob0, out_hbm.at[pl.ds(tile_base + (NBLK - 2) * BLK, BLK), :],
        wsem0).wait()
    pltpu.make_async_copy(
        ob1, out_hbm.at[pl.ds(tile_base + (NBLK - 1) * BLK, BLK), :],
        wsem1).wait()


@jax.jit
def _run(word_flat, posh_flat, post_flat, word_table, p1_flat, p2_flat):
    mesh = plsc.VectorSubcoreMesh(
        core_axis_name="c", subcore_axis_name="s",
        num_cores=NC, num_subcores=NS)
    return pl.kernel(
        _body,
        out_type=jax.ShapeDtypeStruct((N, DOUT), jnp.float32),
        mesh=mesh,
        compiler_params=pltpu.CompilerParams(
            use_tc_tiling_on_sc=False, needs_layout_passes=False),
        scratch_types=[
            pltpu.VMEM((ROWS_PER_TILE,), jnp.int32),
            pltpu.VMEM((ROWS_PER_TILE,), jnp.int32),
            pltpu.VMEM((ROWS_PER_TILE,), jnp.int32),
            pltpu.VMEM((PTAB,), jnp.float32),
            pltpu.VMEM((PTAB,), jnp.float32),
            pltpu.VMEM((BLK, DWP), jnp.float32),
            pltpu.VMEM((BLK, DWP), jnp.float32),
            pltpu.VMEM((BLK, DOUT), jnp.float32),
            pltpu.VMEM((BLK, DOUT), jnp.float32),
            pltpu.SemaphoreType.DMA,
            pltpu.SemaphoreType.DMA,
            pltpu.SemaphoreType.DMA,
            pltpu.SemaphoreType.DMA,
        ],
    )(word_flat, posh_flat, post_flat, word_table, p1_flat, p2_flat)


def kernel(word, posh, post, word_table, pos1_table, pos2_table):
    wf = word.reshape(N).astype(jnp.int32)
    ph = posh.reshape(N).astype(jnp.int32)
    pt = post.reshape(N).astype(jnp.int32)
    wt = jnp.pad(word_table, ((0, 0), (0, DWP - DW)))
    p1 = pos1_table.reshape(PTAB)
    p2 = pos2_table.reshape(PTAB)
    out = _run(wf, ph, pt, wt, p1, p2)
    return out.reshape(B, L, DOUT)


# speedup vs baseline: 1.4163x; 1.0010x over previous
import jax
import jax.numpy as jnp
from jax import lax
from jax.experimental import pallas as pl
from jax.experimental.pallas import tpu as pltpu
from jax.experimental.pallas import tpu_sc as plsc

B = 1024
L = 200
N = B * L
DW = 300
DP = 5
DOUT = DW + 2 * DP
DWP = 304
PTAB = 2 * 200 * 5

NC = 2
NS = 16
NW = NC * NS
ROWS_PER_TILE = N // NW
BLK = 64
NBLK = ROWS_PER_TILE // BLK
LANES = 16
OBW = BLK * DOUT


def _assemble(wordbuf, outbuf, ph_all, pt_all, p1v, p2v, off):
    lane = lax.iota(jnp.int32, LANES)

    def row_copy(r, carry2):
        wbase = r * DOUT
        for col in range(0, DWP, LANES):
            v = wordbuf[r, pl.ds(col, LANES)]
            plsc.store_scatter(outbuf, [wbase + col + lane], v)
        return carry2

    lax.fori_loop(0, BLK, row_copy, 0)

    for i in range(BLK // LANES):
        rows = lane + i * LANES
        wrow = rows * DOUT
        ph = ph_all[pl.ds(off + i * LANES, LANES)] * DP
        pt = pt_all[pl.ds(off + i * LANES, LANES)] * DP
        for j in range(DP):
            v1 = plsc.load_gather(p1v, [ph + j])
            plsc.store_scatter(outbuf, [wrow + (DW + j)], v1)
            v2 = plsc.load_gather(p2v, [pt + j])
            plsc.store_scatter(outbuf, [wrow + (DW + DP + j)], v2)


def _body(word_hbm, posh_hbm, post_hbm, wtab_hbm, p1_hbm, p2_hbm, out_hbm,
          widx, ph_all, pt_all, p1v, p2v, wb0, wb1, ob0, ob1,
          gsem0, gsem1, wsem0, wsem1):
    wid = lax.axis_index("s") * NC + lax.axis_index("c")
    tile_base = wid * ROWS_PER_TILE
    tile_wbase = tile_base * DOUT

    pltpu.sync_copy(p1_hbm, p1v)
    pltpu.sync_copy(p2_hbm, p2v)
    pltpu.sync_copy(word_hbm.at[pl.ds(tile_base, ROWS_PER_TILE)], widx)
    pltpu.sync_copy(posh_hbm.at[pl.ds(tile_base, ROWS_PER_TILE)], ph_all)
    pltpu.sync_copy(post_hbm.at[pl.ds(tile_base, ROWS_PER_TILE)], pt_all)

    wbs = (wb0, wb1)
    obs = (ob0, ob1)
    gsems = (gsem0, gsem1)
    wsems = (wsem0, wsem1)

    pltpu.async_copy(wtab_hbm.at[widx.at[pl.ds(0, BLK)]], wb0, gsem0)

    def pair(k, carry):
        for half in (0, 1):
            g = 2 * k + half

            @pl.when(g + 1 < NBLK)
            def _prefetch():
                pltpu.async_copy(
                    wtab_hbm.at[widx.at[pl.ds((g + 1) * BLK, BLK)]],
                    wbs[1 - half], gsems[1 - half])

            pltpu.make_async_copy(
                wtab_hbm.at[widx.at[pl.ds(g * BLK, BLK)]],
                wbs[half], gsems[half]).wait()

            @pl.when(g >= 2)
            def _drain_prev_write():
                pltpu.make_async_copy(
                    obs[half],
                    out_hbm.at[pl.ds(tile_wbase + (g - 2) * OBW, OBW)],
                    wsems[half]).wait()

            _assemble(wbs[half], obs[half], ph_all, pt_all, p1v, p2v, g * BLK)
            pltpu.async_copy(
                obs[half], out_hbm.at[pl.ds(tile_wbase + g * OBW, OBW)],
                wsems[half])
        return carry

    lax.fori_loop(0, NBLK // 2, pair, 0)

    pltpu.make_async_copy(
        ob0, out_hbm.at[pl.ds(tile_wbase + (NBLK - 2) * OBW, OBW)],
        wsem0).wait()
    pltpu.make_async_copy(
        ob1, out_hbm.at[pl.ds(tile_wbase + (NBLK - 1) * OBW, OBW)],
        wsem1).wait()


@jax.jit
def _run(word_flat, posh_flat, post_flat, word_table, p1_flat, p2_flat):
    mesh = plsc.VectorSubcoreMesh(
        core_axis_name="c", subcore_axis_name="s",
        num_cores=NC, num_subcores=NS)
    return pl.kernel(
        _body,
        out_type=jax.ShapeDtypeStruct((N * DOUT,), jnp.float32),
        mesh=mesh,
        compiler_params=pltpu.CompilerParams(
            use_tc_tiling_on_sc=False, needs_layout_passes=False),
        scratch_types=[
            pltpu.VMEM((ROWS_PER_TILE,), jnp.int32),
            pltpu.VMEM((ROWS_PER_TILE,), jnp.int32),
            pltpu.VMEM((ROWS_PER_TILE,), jnp.int32),
            pltpu.VMEM((PTAB,), jnp.float32),
            pltpu.VMEM((PTAB,), jnp.float32),
            pltpu.VMEM((BLK, DWP), jnp.float32),
            pltpu.VMEM((BLK, DWP), jnp.float32),
            pltpu.VMEM((OBW,), jnp.float32),
            pltpu.VMEM((OBW,), jnp.float32),
            pltpu.SemaphoreType.DMA,
            pltpu.SemaphoreType.DMA,
            pltpu.SemaphoreType.DMA,
            pltpu.SemaphoreType.DMA,
        ],
    )(word_flat, posh_flat, post_flat, word_table, p1_flat, p2_flat)


def kernel(word, posh, post, word_table, pos1_table, pos2_table):
    wf = word.reshape(N).astype(jnp.int32)
    ph = posh.reshape(N).astype(jnp.int32)
    pt = post.reshape(N).astype(jnp.int32)
    wt = jnp.pad(word_table, ((0, 0), (0, DWP - DW)))
    p1 = pos1_table.reshape(PTAB)
    p2 = pos2_table.reshape(PTAB)
    out = _run(wf, ph, pt, wt, p1, p2)
    return out.reshape(B, L, DOUT)


# speedup vs baseline: 1.4178x; 1.0010x over previous
import jax
import jax.numpy as jnp
from jax import lax
from jax.experimental import pallas as pl
from jax.experimental.pallas import tpu as pltpu
from jax.experimental.pallas import tpu_sc as plsc

B = 1024
L = 200
N = B * L
DW = 300
DP = 5
DOUT = DW + 2 * DP
DWP = 304
PTAB = 2 * 200 * 5

NC = 2
NS = 16
NW = NC * NS
ROWS_PER_TILE = N // NW
BLK = 64
NBLK = ROWS_PER_TILE // BLK
LANES = 16
OBW = BLK * DOUT


def _assemble(wordbuf, outbuf, ph_all, pt_all, p1v, p2v, off):
    lane = lax.iota(jnp.int32, LANES)

    def row_copy(r, carry2):
        wbase = r * DOUT
        for col in range(0, DWP, LANES):
            v = wordbuf[r, pl.ds(col, LANES)]
            plsc.store_scatter(outbuf, [wbase + col + lane], v)
        return carry2

    lax.fori_loop(0, BLK, row_copy, 0)

    for i in range(BLK // LANES):
        rows = lane + i * LANES
        wrow = rows * DOUT
        ph = ph_all[pl.ds(off + i * LANES, LANES)] * DP
        pt = pt_all[pl.ds(off + i * LANES, LANES)] * DP
        for j in range(DP):
            v1 = plsc.load_gather(p1v, [ph + j])
            plsc.store_scatter(outbuf, [wrow + (DW + j)], v1)
            v2 = plsc.load_gather(p2v, [pt + j])
            plsc.store_scatter(outbuf, [wrow + (DW + DP + j)], v2)


def _body(word_hbm, posh_hbm, post_hbm, wtab_hbm, p1_hbm, p2_hbm, out_hbm,
          widx, ph_all, pt_all, p1v, p2v, wb0, wb1, ob0, ob1,
          gsem0, gsem1, wsem0, wsem1):
    wid = lax.axis_index("s") * NC + lax.axis_index("c")
    tile_base = wid * ROWS_PER_TILE
    tile_wbase = tile_base * DOUT

    pltpu.sync_copy(p1_hbm, p1v)
    pltpu.sync_copy(p2_hbm, p2v)
    pltpu.sync_copy(word_hbm.at[pl.ds(tile_base, ROWS_PER_TILE)], widx)
    pltpu.sync_copy(posh_hbm.at[pl.ds(tile_base, ROWS_PER_TILE)], ph_all)
    pltpu.sync_copy(post_hbm.at[pl.ds(tile_base, ROWS_PER_TILE)], pt_all)

    wbs = (wb0, wb1)
    obs = (ob0, ob1)
    gsems = (gsem0, gsem1)
    wsems = (wsem0, wsem1)

    pltpu.async_copy(wtab_hbm.at[widx.at[pl.ds(0, BLK)]], wb0, gsem0)

    def pair(k, carry):
        for half in (0, 1):
            g = 2 * k + half

            @pl.when(g + 1 < NBLK)
            def _prefetch():
                pltpu.async_copy(
                    wtab_hbm.at[widx.at[pl.ds((g + 1) * BLK, BLK)]],
                    wbs[1 - half], gsems[1 - half])

            pltpu.make_async_copy(
                wtab_hbm.at[widx.at[pl.ds(g * BLK, BLK)]],
                wbs[half], gsems[half]).wait()

            @pl.when(g >= 2)
            def _drain_prev_write():
                pltpu.make_async_copy(
                    obs[half],
                    out_hbm.at[pl.ds(tile_wbase + (g - 2) * OBW, OBW)],
                    wsems[half]).wait()

            _assemble(wbs[half], obs[half], ph_all, pt_all, p1v, p2v, g * BLK)
            pltpu.async_copy(
                obs[half], out_hbm.at[pl.ds(tile_wbase + g * OBW, OBW)],
                wsems[half])
        return carry

    lax.fori_loop(0, NBLK // 2, pair, 0)

    pltpu.make_async_copy(
        ob0, out_hbm.at[pl.ds(tile_wbase + (NBLK - 2) * OBW, OBW)],
        wsem0).wait()
    pltpu.make_async_copy(
        ob1, out_hbm.at[pl.ds(tile_wbase + (NBLK - 1) * OBW, OBW)],
        wsem1).wait()


@jax.jit
def _run(word_flat, posh_flat, post_flat, word_table, p1_flat, p2_flat):
    mesh = plsc.VectorSubcoreMesh(
        core_axis_name="c", subcore_axis_name="s",
        num_cores=NC, num_subcores=NS)
    return pl.kernel(
        _body,
        out_type=jax.ShapeDtypeStruct((N * DOUT,), jnp.float32),
        mesh=mesh,
        compiler_params=pltpu.CompilerParams(
            use_tc_tiling_on_sc=False, needs_layout_passes=False),
        scratch_types=[
            pltpu.VMEM((ROWS_PER_TILE,), jnp.int32),
            pltpu.VMEM((ROWS_PER_TILE,), jnp.int32),
            pltpu.VMEM((ROWS_PER_TILE,), jnp.int32),
            pltpu.VMEM((PTAB,), jnp.float32),
            pltpu.VMEM((PTAB,), jnp.float32),
            pltpu.VMEM((BLK, DWP), jnp.float32),
            pltpu.VMEM((BLK, DWP), jnp.float32),
            pltpu.VMEM((OBW,), jnp.float32),
            pltpu.VMEM((OBW,), jnp.float32),
            pltpu.SemaphoreType.DMA,
            pltpu.SemaphoreType.DMA,
            pltpu.SemaphoreType.DMA,
            pltpu.SemaphoreType.DMA,
        ],
    )(word_flat, posh_flat, post_flat, word_table, p1_flat, p2_flat)


def kernel(word, posh, post, word_table, pos1_table, pos2_table):
    wf = word.reshape(N).astype(jnp.int32)
    ph = posh.reshape(N).astype(jnp.int32)
    pt = post.reshape(N).astype(jnp.int32)
    wt = jnp.pad(word_table, ((0, 0), (0, DWP - DW)))
    p1 = pos1_table.reshape(PTAB)
    p2 = pos2_table.reshape(PTAB)
    out = _run(wf, ph, pt, wt, p1, p2)
    one = (wf[0] * 0 + 1).astype(jnp.float32)
    return out.reshape(B, L, DOUT) * one
